# baseline (speedup 1.0000x reference)
import jax
import jax.numpy as jnp
from jax.experimental import pallas as pl
from jax.experimental.pallas import tpu as pltpu

TM_MAX = 8192


def _round_up(n, m):
    return ((n + m - 1) // m) * m


def _mlp_body(xa_ref, xb_ref, w1_ref, b1_ref, w2t_ref, b2_ref, w3_ref, b3_ref,
              ot_ref):
    b1c = jnp.transpose(b1_ref[...])
    b2c = jnp.transpose(b2_ref[...])
    b3c = jnp.transpose(b3_ref[...])
    hm = xa_ref.shape[0]
    for s, x_ref in enumerate((xa_ref, xb_ref)):
        h = jax.lax.dot_general(w1_ref[...], x_ref[...],
                                (((0,), (1,)), ((), ())),
                                preferred_element_type=jnp.float32)
        h = jnp.maximum(h + b1c, 0.0)
        h = jax.lax.dot_general(w2t_ref[...], h, (((1,), (0,)), ((), ())),
                                preferred_element_type=jnp.float32)
        h = jnp.maximum(h + b2c, 0.0)
        ot = jax.lax.dot_general(w3_ref[...], h, (((0,), (0,)), ((), ())),
                                 preferred_element_type=jnp.float32)
        ot_ref[:, pl.ds(s * hm, hm)] = ot + b3c


def kernel(x, w1, b1, w2, b2, w3, b3):
    B, in_size = x.shape
    out_size = w3.shape[1]

    tm = min(TM_MAX, _round_up(B, 8))
    b_pad = _round_up(B, tm)
    xp = jnp.pad(x, ((0, b_pad - B), (0, 0))) if b_pad != B else x
    grid = (b_pad // tm,)

    w2t = w2.T

    def const_spec(a):
        return pl.BlockSpec(a.shape, lambda i: (0,) * a.ndim)

    flops = 2 * b_pad * (in_size * w1.shape[1] + w1.shape[1] * w2.shape[1]
                         + w2.shape[1] * out_size)
    bytes_accessed = 4 * (b_pad * in_size + b_pad * out_size
                          + w1.size + b1.size + w2.size + b2.size
                          + w3.size + b3.size)

    hm = tm // 2
    out_t = pl.pallas_call(
        _mlp_body,
        out_shape=jax.ShapeDtypeStruct((out_size, b_pad), jnp.float32),
        grid=grid,
        in_specs=[
            pl.BlockSpec((hm, in_size), lambda i: (2 * i, 0)),
            pl.BlockSpec((hm, in_size), lambda i: (2 * i + 1, 0)),
            const_spec(w1), const_spec(b1),
            const_spec(w2t), const_spec(b2),
            const_spec(w3), const_spec(b3),
        ],
        out_specs=pl.BlockSpec((out_size, tm), lambda i: (0, i)),
        compiler_params=pltpu.CompilerParams(
            dimension_semantics=("parallel",),
        ),
        cost_estimate=pl.CostEstimate(
            flops=flops, transcendentals=0, bytes_accessed=bytes_accessed),
    )(xp, xp, w1, b1, w2t, b2, w3, b3)

    out = out_t.T
    return out[:B] if b_pad != B else out


# speedup vs baseline: 1.0046x; 1.0046x over previous
import jax
import jax.numpy as jnp
from jax.experimental import pallas as pl
from jax.experimental.pallas import tpu as pltpu

TM_MAX = 8192


def _round_up(n, m):
    return ((n + m - 1) // m) * m


def _mlp_body(x_ref, w1_ref, b1_ref, w2t_ref, b2_ref, w3_ref, b3_ref, ot_ref):
    h = jax.lax.dot_general(w1_ref[...], x_ref[...], (((0,), (1,)), ((), ())),
                            preferred_element_type=jnp.float32)
    h = jnp.maximum(h + jnp.transpose(b1_ref[...]), 0.0)
    h = jax.lax.dot_general(w2t_ref[...], h, (((1,), (0,)), ((), ())),
                            preferred_element_type=jnp.float32)
    h = jnp.maximum(h + jnp.transpose(b2_ref[...]), 0.0)
    ot = jax.lax.dot_general(w3_ref[...], h, (((0,), (0,)), ((), ())),
                             preferred_element_type=jnp.float32)
    ot_ref[...] = ot + jnp.transpose(b3_ref[...])


def kernel(x, w1, b1, w2, b2, w3, b3):
    B, in_size = x.shape
    out_size = w3.shape[1]

    tm = min(TM_MAX, _round_up(B, 8))
    b_pad = _round_up(B, tm)
    xp = jnp.pad(x, ((0, b_pad - B), (0, 0))) if b_pad != B else x
    grid = (b_pad // tm,)

    w2t = w2.T

    def const_spec(a):
        return pl.BlockSpec(a.shape, lambda i: (0,) * a.ndim)

    flops = 2 * b_pad * (in_size * w1.shape[1] + w1.shape[1] * w2.shape[1]
                         + w2.shape[1] * out_size)
    bytes_accessed = 4 * (b_pad * in_size + b_pad * out_size
                          + w1.size + b1.size + w2.size + b2.size
                          + w3.size + b3.size)

    out_t = pl.pallas_call(
        _mlp_body,
        out_shape=jax.ShapeDtypeStruct((out_size, b_pad), jnp.float32),
        grid=grid,
        in_specs=[
            pl.BlockSpec((tm, in_size), lambda i: (i, 0)),
            const_spec(w1), const_spec(b1),
            const_spec(w2t), const_spec(b2),
            const_spec(w3), const_spec(b3),
        ],
        out_specs=pl.BlockSpec((out_size, tm), lambda i: (0, i)),
        compiler_params=pltpu.CompilerParams(
            dimension_semantics=("parallel",),
        ),
        cost_estimate=pl.CostEstimate(
            flops=flops, transcendentals=0, bytes_accessed=bytes_accessed),
    )(xp, w1, b1, w2t, b2, w3, b3)

    out = out_t.T
    return out[:B] if b_pad != B else out
